# Initial kernel scaffold; baseline (speedup 1.0000x reference)
import functools

import jax
import jax.numpy as jnp
from jax import lax
from jax.experimental import pallas as pl
from jax.experimental.pallas import tpu as pltpu
from jax.experimental.pallas import tpu_sc as plsc

_N = 10000
_E = 320000
_EMB = 128
_NP = 10240
_DUMMY = _NP - 1
_NC = 2
_NS = 16
_NW = _NC * _NS
_TB = 158
_DEG_B = 79
_DEGW = 16
_MBLK = 1024
_MGRID = _NP // _MBLK
_RB = _MBLK // 128

_mesh = plsc.VectorSubcoreMesh(core_axis_name="c", subcore_axis_name="s",
                               num_cores=_NC, num_subcores=_NS)


@functools.partial(
    pl.kernel,
    out_type=[
        jax.ShapeDtypeStruct((_NP, _EMB), jnp.float32),
        jax.ShapeDtypeStruct((_NC, _NP, _DEGW), jnp.float32),
    ],
    mesh=_mesh,
    scratch_types=[
        pltpu.VMEM((3, 128), jnp.int32),
        pltpu.VMEM((128, _EMB), jnp.float32),
        pltpu.VMEM((_DEG_B, 128), jnp.int32),
        pltpu.VMEM((128, _DEGW), jnp.float32),
        pltpu.VMEM((128, _DEGW), jnp.float32),
        pltpu.VMEM_SHARED((_NP, _DEGW), jnp.float32),
        pltpu.SemaphoreType.DMA,
    ],
)
def _emb_deg(emb_hbm, tipo_hbm, dstd_hbm, xg_hbm, degp_hbm,
             idxv, rows, dstv, ones, zeros, deg_sh, sem):
    c = lax.axis_index("c")
    s = lax.axis_index("s")
    w = s * _NC + c

    @pl.loop(0, 128)
    def _init(i):
        ones[i, :] = jnp.ones((_DEGW,), jnp.float32)
        zeros[i, :] = jnp.zeros((_DEGW,), jnp.float32)

    @pl.loop(0, 5)
    def _zero(k):
        pltpu.sync_copy(zeros, deg_sh.at[pl.ds(s * 640 + k * 128, 128)])

    pltpu.sync_copy(tipo_hbm.at[w], idxv)
    pltpu.sync_copy(dstd_hbm.at[c, s], dstv)

    @pl.loop(0, 3)
    def _gather(k):
        base = w * 384 + k * 128

        @pl.when(base < _NP)
        def _():
            pltpu.async_copy(emb_hbm.at[idxv.at[k]], rows, sem).wait()
            pltpu.sync_copy(rows, xg_hbm.at[pl.ds(base, 128)])

    plsc.subcore_barrier()

    @pl.loop(0, _DEG_B)
    def _deg(j):
        pltpu.sync_copy(ones, deg_sh.at[dstv.at[j]], add=True)

    plsc.subcore_barrier()
    pltpu.sync_copy(deg_sh.at[pl.ds(s * 640, 640)],
                    degp_hbm.at[c, pl.ds(s * 640, 640)])


@functools.partial(
    pl.kernel,
    out_type=jax.ShapeDtypeStruct((_NC, _NP, 128), jnp.float32),
    mesh=_mesh,
    scratch_types=[
        pltpu.VMEM((_TB, 128), jnp.int32),
        pltpu.VMEM((_TB, 128), jnp.int32),
        pltpu.VMEM((128, 128), jnp.float32),
        pltpu.VMEM_SHARED((_NP, 128), jnp.float32),
        pltpu.SemaphoreType.DMA,
    ],
)
def _prop(zflat_hbm, srcv_hbm, dstv_hbm, acc_hbm,
          srcv, dstv, buf0, acc_sh, sem0):
    c = lax.axis_index("c")
    s = lax.axis_index("s")

    @pl.loop(0, 128)
    def _zb(i):
        for j in range(8):
            buf0[i, pl.ds(j * 16, 16)] = jnp.zeros((16,), jnp.float32)

    @pl.loop(0, 5)
    def _za(k):
        pltpu.sync_copy(buf0, acc_sh.at[pl.ds(s * 640 + k * 128, 128)])

    pltpu.sync_copy(srcv_hbm.at[c, s], srcv)
    pltpu.sync_copy(dstv_hbm.at[s], dstv)
    plsc.subcore_barrier()

    @pl.loop(0, _TB)
    def _main(j):
        pltpu.async_copy(zflat_hbm.at[srcv.at[j]], buf0, sem0).wait()
        pltpu.sync_copy(buf0, acc_sh.at[dstv.at[j]], add=True)

    plsc.subcore_barrier()
    pltpu.sync_copy(acc_sh.at[pl.ds(s * 640, 640)],
                    acc_hbm.at[c, pl.ds(s * 640, 640)])


def _scale_body(degp_ref, mask_ref, dinv_ref, rs1_ref):
    deg = degp_ref[0] + degp_ref[1] + 1.0
    dinv = lax.rsqrt(deg)
    dinv_ref[...] = dinv
    rs1_ref[...] = dinv * mask_ref[...]


_scale = pl.pallas_call(
    _scale_body,
    out_shape=[jax.ShapeDtypeStruct((80, 128), jnp.float32),
               jax.ShapeDtypeStruct((80, 128), jnp.float32)],
)


def _rowscale(y, rs):
    return (y.reshape(_RB, 128, 128) * rs[:, :, None]).reshape(_MBLK, 128)


def _mm1_body(x_ref, w_ref, rs_ref, z_ref):
    y = jnp.dot(x_ref[...], w_ref[...], preferred_element_type=jnp.float32,
                precision=lax.Precision.HIGHEST)
    z_ref[...] = _rowscale(y, rs_ref[...])


_mm1 = pl.pallas_call(
    _mm1_body,
    grid=(_MGRID, 2),
    in_specs=[
        pl.BlockSpec((_MBLK, _EMB), lambda i, j: (i, 0)),
        pl.BlockSpec((_EMB, 128), lambda i, j: (0, j)),
        pl.BlockSpec((_RB, 128), lambda i, j: (i, 0)),
    ],
    out_specs=pl.BlockSpec((None, _MBLK, 128), lambda i, j: (j, i, 0)),
    out_shape=jax.ShapeDtypeStruct((_NC, _NP, 128), jnp.float32),
)


def _mm2_body(acc_ref, z_ref, dinv_ref, w_ref, out_ref):
    k = pl.program_id(2)
    dinv = dinv_ref[...]
    h = _rowscale(acc_ref[...] + z_ref[...], dinv)
    h = jnp.maximum(h, 0.0)
    y = jnp.dot(h, w_ref[...], preferred_element_type=jnp.float32,
                precision=lax.Precision.HIGHEST)

    @pl.when(k == 0)
    def _():
        out_ref[...] = y

    @pl.when(k == 1)
    def _():
        out_ref[...] = _rowscale(out_ref[...] + y, dinv)


_mm2 = pl.pallas_call(
    _mm2_body,
    grid=(_MGRID, 2, 2),
    in_specs=[
        pl.BlockSpec((None, _MBLK, 128), lambda i, j, k: (k, i, 0)),
        pl.BlockSpec((None, _MBLK, 128), lambda i, j, k: (k, i, 0)),
        pl.BlockSpec((_RB, 128), lambda i, j, k: (i, 0)),
        pl.BlockSpec((128, 128), lambda i, j, k: (k, j)),
    ],
    out_specs=pl.BlockSpec((None, _MBLK, 128), lambda i, j, k: (j, i, 0)),
    out_shape=jax.ShapeDtypeStruct((_NC, _NP, 128), jnp.float32),
)


def _fin_body(acc_ref, z_ref, dinv_ref, out_ref):
    out_ref[...] = _rowscale(acc_ref[...] + z_ref[...], dinv_ref[...])


_fin = pl.pallas_call(
    _fin_body,
    grid=(_MGRID, 2),
    in_specs=[
        pl.BlockSpec((None, _MBLK, 128), lambda i, j: (j, i, 0)),
        pl.BlockSpec((None, _MBLK, 128), lambda i, j: (j, i, 0)),
        pl.BlockSpec((_RB, 128), lambda i, j: (i, 0)),
    ],
    out_specs=pl.BlockSpec((_MBLK, 128), lambda i, j: (i, j)),
    out_shape=jax.ShapeDtypeStruct((_NP, 256), jnp.float32),
)


def kernel(edge_index, tipo_ids, mask_embed, emb_table, W1, W2):
    src = edge_index[0]
    dst = edge_index[1]

    perw = _E // _NS
    pad = _TB * 128 - perw
    srcp = jnp.pad(src.reshape(_NS, perw), ((0, 0), (0, pad))) \
              .reshape(_NS, _TB, 128)
    srcv_all = jnp.stack([srcp, srcp + _NP])
    dstp = jnp.pad(dst.reshape(_NS, perw), ((0, 0), (0, pad)),
                   constant_values=_DUMMY).reshape(_NS, _TB, 128)
    perd = _E // _NW
    dpad = _DEG_B * 128 - perd
    dstd = jnp.pad(dst.reshape(_NC, _NS, perd), ((0, 0), (0, 0), (0, dpad)),
                   constant_values=_DUMMY).reshape(_NC, _NS, _DEG_B, 128)
    tipo3 = jnp.pad(tipo_ids, (0, _NW * 384 - _N)).reshape(_NW, 3, 128)
    mask2d = jnp.pad(mask_embed, (0, _NP - _N)).reshape(80, 128)

    xg, degp = _emb_deg(emb_table, tipo3, dstd)
    dinv2d, rs1 = _scale(degp[:, :, 0].reshape(_NC, 80, 128), mask2d)
    z1 = _mm1(xg, W1, rs1)
    acc1 = _prop(z1.reshape(_NC * _NP, 128), srcv_all, dstp)
    z2 = _mm2(acc1, z1, dinv2d, W2)
    acc2 = _prop(z2.reshape(_NC * _NP, 128), srcv_all, dstp)
    out = _fin(acc2, z2, dinv2d)
    return out[:_N]


# speedup vs baseline: 4.9621x; 4.9621x over previous
import functools

import jax
import jax.numpy as jnp
from jax import lax
from jax.experimental import pallas as pl
from jax.experimental.pallas import tpu as pltpu
from jax.experimental.pallas import tpu_sc as plsc

_N = 10000
_E = 320000
_EMB = 128
_NP = 10240
_DUMMY = _NP - 1
_NC = 2
_NS = 16
_NW = _NC * _NS
_TB = 158
_HR = _NP // 2
_AR = 5376
_LDUMMY = _AR - 1
_DEG_B = 79
_DEGW = 16
_MBLK = 1024
_MGRID = _NP // _MBLK
_RB = _MBLK // 128

def _emb_deg_body(emb_hbm, tipo_hbm, dstd_hbm, xg_hbm, degp_hbm,
                  idxv, rows, dstv, hist, sem):
    c = lax.axis_index("c")
    s = lax.axis_index("s")
    w = s * _NC + c

    @pl.loop(0, _NP // 16)
    def _zh(i):
        hist[pl.ds(i * 16, 16)] = jnp.zeros((16,), jnp.float32)

    pltpu.sync_copy(tipo_hbm.at[w], idxv)
    pltpu.sync_copy(dstd_hbm.at[w], dstv)

    @pl.loop(0, 3)
    def _gather(k):
        pltpu.async_copy(emb_hbm.at[idxv.at[k]], rows, sem).wait()
        pltpu.sync_copy(rows, xg_hbm.at[pl.ds(w * 384 + k * 128, 128)])

    @pl.loop(0, _E // _NW // 16)
    def _dh(j):
        idx = dstv[pl.ds(j * 16, 16)]
        plsc.addupdate_scatter(hist, [idx], jnp.ones((16,), jnp.float32))

    pltpu.sync_copy(hist, degp_hbm.at[w])


def _prop_body(zflat_hbm, srcv_hbm, dstv_hbm, acc_hbm,
               srcv, dstv, buf0, zbuf, acc_sh, sem0):
    c = lax.axis_index("c")
    s = lax.axis_index("s")

    @pl.loop(0, 128)
    def _zb(i):
        for j in range(8):
            zbuf[i, pl.ds(j * 16, 16)] = jnp.zeros((16,), jnp.float32)

    pltpu.sync_copy(srcv_hbm.at[c, s], srcv)

    @pl.loop(0, 2)
    def _pass(r):
        zb = s * 336
        pltpu.sync_copy(zbuf, acc_sh.at[pl.ds(zb, 128)])
        pltpu.sync_copy(zbuf, acc_sh.at[pl.ds(zb + 128, 128)])
        pltpu.sync_copy(zbuf.at[pl.ds(0, 80)], acc_sh.at[pl.ds(zb + 256, 80)])
        pltpu.sync_copy(dstv_hbm.at[r, s], dstv)
        plsc.subcore_barrier()

        @pl.loop(0, _TB)
        def _main(j):
            pltpu.async_copy(zflat_hbm.at[srcv.at[j]], buf0, sem0).wait()
            pltpu.sync_copy(buf0, acc_sh.at[dstv.at[j]], add=True)

        plsc.subcore_barrier()
        pltpu.sync_copy(acc_sh.at[pl.ds(s * 320, 320)],
                        acc_hbm.at[c, pl.ds(r * _HR + s * 320, 320)])
        plsc.subcore_barrier()


@functools.cache
def _sc_kernels():
    mesh = plsc.VectorSubcoreMesh(core_axis_name="c", subcore_axis_name="s",
                                  num_cores=_NC, num_subcores=_NS)
    emb_deg = pl.kernel(
        _emb_deg_body,
        compiler_params=pltpu.CompilerParams(needs_layout_passes=False),
        out_type=[
            jax.ShapeDtypeStruct((_NW * 384, _EMB), jnp.float32),
            jax.ShapeDtypeStruct((_NW, _NP), jnp.float32),
        ],
        mesh=mesh,
        scratch_types=[
            pltpu.VMEM((3, 128), jnp.int32),
            pltpu.VMEM((128, _EMB), jnp.float32),
            pltpu.VMEM((_E // _NW,), jnp.int32),
            pltpu.VMEM((_NP,), jnp.float32),
            pltpu.SemaphoreType.DMA,
        ],
    )
    prop = pl.kernel(
        _prop_body,
        out_type=jax.ShapeDtypeStruct((_NC, _NP, 128), jnp.float32),
        mesh=mesh,
        scratch_types=[
            pltpu.VMEM((_TB, 128), jnp.int32),
            pltpu.VMEM((_TB, 128), jnp.int32),
            pltpu.VMEM((128, 128), jnp.float32),
            pltpu.VMEM((128, 128), jnp.float32),
            pltpu.VMEM_SHARED((_AR, 128), jnp.float32),
            pltpu.SemaphoreType.DMA,
        ],
    )
    return emb_deg, prop


def _scale_body(degp_ref, mask_ref, dinv_ref, rs1_ref):
    deg = jnp.sum(degp_ref[...], axis=0) + 1.0
    dinv = lax.rsqrt(deg)
    dinv_ref[...] = dinv
    rs1_ref[...] = dinv * mask_ref[...]


_scale = pl.pallas_call(
    _scale_body,
    out_shape=[jax.ShapeDtypeStruct((80, 128), jnp.float32),
               jax.ShapeDtypeStruct((80, 128), jnp.float32)],
)


def _rowscale(y, rs):
    w = y.shape[-1]
    return (y.reshape(_RB, 128, w) * rs[:, :, None]).reshape(_MBLK, w)


def _mm1_body(x_ref, w_ref, rs_ref, z_ref):
    y = jnp.dot(x_ref[...], w_ref[...], preferred_element_type=jnp.float32,
                precision=lax.Precision.HIGHEST)
    z_ref[...] = _rowscale(y, rs_ref[...])


_mm1 = pl.pallas_call(
    _mm1_body,
    grid=(_MGRID, 2),
    in_specs=[
        pl.BlockSpec((_MBLK, _EMB), lambda i, j: (i, 0)),
        pl.BlockSpec((_EMB, 128), lambda i, j: (0, j)),
        pl.BlockSpec((_RB, 128), lambda i, j: (i, 0)),
    ],
    out_specs=pl.BlockSpec((None, _MBLK, 128), lambda i, j: (j, i, 0)),
    out_shape=jax.ShapeDtypeStruct((_NC, _NP, 128), jnp.float32),
)


def _mm2_body(acc_ref, z_ref, dinv_ref, w_ref, out_ref):
    k = pl.program_id(2)
    dinv = dinv_ref[...]
    h = _rowscale(acc_ref[...] + z_ref[...], dinv)
    h = jnp.maximum(h, 0.0)
    y = jnp.dot(h, w_ref[...], preferred_element_type=jnp.float32,
                precision=lax.Precision.HIGHEST)

    @pl.when(k == 0)
    def _():
        out_ref[...] = y

    @pl.when(k == 1)
    def _():
        out_ref[...] = _rowscale(out_ref[...] + y, dinv)


_mm2 = pl.pallas_call(
    _mm2_body,
    grid=(_MGRID, 2, 2),
    in_specs=[
        pl.BlockSpec((None, _MBLK, 128), lambda i, j, k: (k, i, 0)),
        pl.BlockSpec((None, _MBLK, 128), lambda i, j, k: (k, i, 0)),
        pl.BlockSpec((_RB, 128), lambda i, j, k: (i, 0)),
        pl.BlockSpec((128, 128), lambda i, j, k: (k, j)),
    ],
    out_specs=pl.BlockSpec((None, _MBLK, 128), lambda i, j, k: (j, i, 0)),
    out_shape=jax.ShapeDtypeStruct((_NC, _NP, 128), jnp.float32),
)


def _fin_body(acc_ref, z_ref, dinv_ref, out_ref):
    out_ref[...] = _rowscale(acc_ref[...] + z_ref[...], dinv_ref[...])


_fin = pl.pallas_call(
    _fin_body,
    grid=(_MGRID, 2),
    in_specs=[
        pl.BlockSpec((None, _MBLK, 128), lambda i, j: (j, i, 0)),
        pl.BlockSpec((None, _MBLK, 128), lambda i, j: (j, i, 0)),
        pl.BlockSpec((_RB, 128), lambda i, j: (i, 0)),
    ],
    out_specs=pl.BlockSpec((_MBLK, 128), lambda i, j: (i, j)),
    out_shape=jax.ShapeDtypeStruct((_NP, 256), jnp.float32),
)


def kernel(edge_index, tipo_ids, mask_embed, emb_table, W1, W2):
    src = edge_index[0]
    dst = edge_index[1]

    perw = _E // _NS
    pad = _TB * 128 - perw
    srcp = jnp.pad(src.reshape(_NS, perw), ((0, 0), (0, pad))) \
              .reshape(_NS, _TB, 128)
    srcv_all = jnp.stack([srcp, srcp + _NP])
    dstp = jnp.pad(dst.reshape(_NS, perw), ((0, 0), (0, pad)),
                   constant_values=_DUMMY).reshape(_NS, _TB, 128)
    dstp_both = jnp.stack(
        [jnp.where((dstp >= r * _HR) & (dstp < (r + 1) * _HR),
                   dstp - r * _HR, _LDUMMY) for r in range(2)])
    tipo3 = jnp.pad(tipo_ids, (0, _NW * 384 - _N)).reshape(_NW, 3, 128)
    mask2d = jnp.pad(mask_embed, (0, _NP - _N)).reshape(80, 128)

    _emb_deg, _prop = _sc_kernels()
    dstd = dst.reshape(_NW, _E // _NW)
    xg, degp = _emb_deg(emb_table, tipo3, dstd)
    xg = xg[: _NP]
    dinv2d, rs1 = _scale(degp.reshape(_NW, 80, 128), mask2d)
    z1 = _mm1(xg, W1, rs1)
    acc1 = _prop(z1.reshape(_NC * _NP, 128), srcv_all, dstp_both)
    z2 = _mm2(acc1, z1, dinv2d, W2)
    acc2 = _prop(z2.reshape(_NC * _NP, 128), srcv_all, dstp_both)
    out = _fin(acc2, z2, dinv2d)
    return out[:_N]


# speedup vs baseline: 5.6973x; 1.1482x over previous
import functools

import jax
import jax.numpy as jnp
from jax import lax
from jax.experimental import pallas as pl
from jax.experimental.pallas import tpu as pltpu
from jax.experimental.pallas import tpu_sc as plsc

_N = 10000
_E = 320000
_EMB = 128
_NP = 10240
_DUMMY = _NP - 1
_NC = 2
_NS = 16
_NW = _NC * _NS
_TB = 158
_HR = _NP // 2
_AR = 5376
_LDUMMY = _AR - 1
_DEG_B = 79
_DEGW = 16
_MBLK = 1024
_MGRID = _NP // _MBLK
_RB = _MBLK // 128

def _emb_deg_body(emb_hbm, tipo_hbm, dstd_hbm, xg_hbm, degp_hbm,
                  idxv, rows, dstv, hist, sem):
    c = lax.axis_index("c")
    s = lax.axis_index("s")
    w = s * _NC + c

    @pl.loop(0, _NP // 16)
    def _zh(i):
        hist[pl.ds(i * 16, 16)] = jnp.zeros((16,), jnp.float32)

    pltpu.sync_copy(tipo_hbm.at[w], idxv)
    pltpu.sync_copy(dstd_hbm.at[w], dstv)

    @pl.loop(0, 3)
    def _gather(k):
        pltpu.async_copy(emb_hbm.at[idxv.at[k]], rows, sem).wait()
        pltpu.sync_copy(rows, xg_hbm.at[pl.ds(w * 384 + k * 128, 128)])

    @pl.loop(0, _E // _NW // 16)
    def _dh(j):
        idx = dstv[pl.ds(j * 16, 16)]
        plsc.addupdate_scatter(hist, [idx], jnp.ones((16,), jnp.float32))

    pltpu.sync_copy(hist, degp_hbm.at[w])


def _prop_body(zflat_hbm, srcv_hbm, dstv_hbm, acc_hbm,
               srcv, dstv, buf0, buf1, acc_sh, sem0, sem1):
    c = lax.axis_index("c")
    s = lax.axis_index("s")

    pltpu.sync_copy(srcv_hbm.at[c, s], srcv)

    @pl.loop(0, 2)
    def _pass(r):
        @pl.loop(0, 128)
        def _zb(i):
            for j in range(8):
                buf0[i, pl.ds(j * 16, 16)] = jnp.zeros((16,), jnp.float32)

        zb = s * 336
        pltpu.sync_copy(buf0, acc_sh.at[pl.ds(zb, 128)])
        pltpu.sync_copy(buf0, acc_sh.at[pl.ds(zb + 128, 128)])
        pltpu.sync_copy(buf0.at[pl.ds(0, 80)], acc_sh.at[pl.ds(zb + 256, 80)])
        pltpu.sync_copy(dstv_hbm.at[r, s], dstv)
        plsc.subcore_barrier()

        pltpu.async_copy(zflat_hbm.at[srcv.at[0]], buf0, sem0)

        @pl.loop(0, _TB // 2)
        def _main(h):
            j0 = 2 * h
            pltpu.async_copy(zflat_hbm.at[srcv.at[j0 + 1]], buf1, sem1)
            pltpu.make_async_copy(zflat_hbm.at[srcv.at[j0]], buf0, sem0).wait()
            pltpu.sync_copy(buf0, acc_sh.at[dstv.at[j0]], add=True)

            @pl.when(j0 + 2 < _TB)
            def _():
                pltpu.async_copy(zflat_hbm.at[srcv.at[j0 + 2]], buf0, sem0)

            pltpu.make_async_copy(zflat_hbm.at[srcv.at[j0 + 1]], buf1, sem1).wait()
            pltpu.sync_copy(buf1, acc_sh.at[dstv.at[j0 + 1]], add=True)

        plsc.subcore_barrier()
        pltpu.sync_copy(acc_sh.at[pl.ds(s * 320, 320)],
                        acc_hbm.at[c, pl.ds(r * _HR + s * 320, 320)])
        plsc.subcore_barrier()


@functools.cache
def _sc_kernels():
    mesh = plsc.VectorSubcoreMesh(core_axis_name="c", subcore_axis_name="s",
                                  num_cores=_NC, num_subcores=_NS)
    emb_deg = pl.kernel(
        _emb_deg_body,
        compiler_params=pltpu.CompilerParams(needs_layout_passes=False),
        out_type=[
            jax.ShapeDtypeStruct((_NW * 384, _EMB), jnp.float32),
            jax.ShapeDtypeStruct((_NW, _NP), jnp.float32),
        ],
        mesh=mesh,
        scratch_types=[
            pltpu.VMEM((3, 128), jnp.int32),
            pltpu.VMEM((128, _EMB), jnp.float32),
            pltpu.VMEM((_E // _NW,), jnp.int32),
            pltpu.VMEM((_NP,), jnp.float32),
            pltpu.SemaphoreType.DMA,
        ],
    )
    prop = pl.kernel(
        _prop_body,
        out_type=jax.ShapeDtypeStruct((_NC, _NP, 128), jnp.float32),
        mesh=mesh,
        scratch_types=[
            pltpu.VMEM((_TB, 128), jnp.int32),
            pltpu.VMEM((_TB, 128), jnp.int32),
            pltpu.VMEM((128, 128), jnp.float32),
            pltpu.VMEM((128, 128), jnp.float32),
            pltpu.VMEM_SHARED((_AR, 128), jnp.float32),
            pltpu.SemaphoreType.DMA,
            pltpu.SemaphoreType.DMA,
        ],
    )
    return emb_deg, prop


def _scale_body(degp_ref, mask_ref, dinv_ref, rs1_ref):
    deg = jnp.sum(degp_ref[...], axis=0) + 1.0
    dinv = lax.rsqrt(deg)
    dinv_ref[...] = dinv
    rs1_ref[...] = dinv * mask_ref[...]


_scale = pl.pallas_call(
    _scale_body,
    out_shape=[jax.ShapeDtypeStruct((80, 128), jnp.float32),
               jax.ShapeDtypeStruct((80, 128), jnp.float32)],
)


def _rowscale(y, rs):
    w = y.shape[-1]
    return (y.reshape(_RB, 128, w) * rs[:, :, None]).reshape(_MBLK, w)


def _mm1_body(x_ref, w_ref, rs_ref, z_ref):
    y = jnp.dot(x_ref[...], w_ref[...], preferred_element_type=jnp.float32,
                precision=lax.Precision.HIGHEST)
    z_ref[...] = _rowscale(y, rs_ref[...])


_mm1 = pl.pallas_call(
    _mm1_body,
    grid=(_MGRID, 2),
    in_specs=[
        pl.BlockSpec((_MBLK, _EMB), lambda i, j: (i, 0)),
        pl.BlockSpec((_EMB, 128), lambda i, j: (0, j)),
        pl.BlockSpec((_RB, 128), lambda i, j: (i, 0)),
    ],
    out_specs=pl.BlockSpec((None, _MBLK, 128), lambda i, j: (j, i, 0)),
    out_shape=jax.ShapeDtypeStruct((_NC, _NP, 128), jnp.float32),
)


def _mm2_body(acc_ref, z_ref, dinv_ref, w_ref, out_ref):
    k = pl.program_id(2)
    dinv = dinv_ref[...]
    h = _rowscale(acc_ref[...] + z_ref[...], dinv)
    h = jnp.maximum(h, 0.0)
    y = jnp.dot(h, w_ref[...], preferred_element_type=jnp.float32,
                precision=lax.Precision.HIGHEST)

    @pl.when(k == 0)
    def _():
        out_ref[...] = y

    @pl.when(k == 1)
    def _():
        out_ref[...] = _rowscale(out_ref[...] + y, dinv)


_mm2 = pl.pallas_call(
    _mm2_body,
    grid=(_MGRID, 2, 2),
    in_specs=[
        pl.BlockSpec((None, _MBLK, 128), lambda i, j, k: (k, i, 0)),
        pl.BlockSpec((None, _MBLK, 128), lambda i, j, k: (k, i, 0)),
        pl.BlockSpec((_RB, 128), lambda i, j, k: (i, 0)),
        pl.BlockSpec((128, 128), lambda i, j, k: (k, j)),
    ],
    out_specs=pl.BlockSpec((None, _MBLK, 128), lambda i, j, k: (j, i, 0)),
    out_shape=jax.ShapeDtypeStruct((_NC, _NP, 128), jnp.float32),
)


def _fin_body(acc_ref, z_ref, dinv_ref, out_ref):
    out_ref[...] = _rowscale(acc_ref[...] + z_ref[...], dinv_ref[...])


_fin = pl.pallas_call(
    _fin_body,
    grid=(_MGRID, 2),
    in_specs=[
        pl.BlockSpec((None, _MBLK, 128), lambda i, j: (j, i, 0)),
        pl.BlockSpec((None, _MBLK, 128), lambda i, j: (j, i, 0)),
        pl.BlockSpec((_RB, 128), lambda i, j: (i, 0)),
    ],
    out_specs=pl.BlockSpec((_MBLK, 128), lambda i, j: (i, j)),
    out_shape=jax.ShapeDtypeStruct((_NP, 256), jnp.float32),
)


def kernel(edge_index, tipo_ids, mask_embed, emb_table, W1, W2):
    src = edge_index[0]
    dst = edge_index[1]

    perw = _E // _NS
    pad = _TB * 128 - perw
    srcp = jnp.pad(src.reshape(_NS, perw), ((0, 0), (0, pad))) \
              .reshape(_NS, _TB, 128)
    srcv_all = jnp.stack([srcp, srcp + _NP])
    dstp = jnp.pad(dst.reshape(_NS, perw), ((0, 0), (0, pad)),
                   constant_values=_DUMMY).reshape(_NS, _TB, 128)
    dstp_both = jnp.stack(
        [jnp.where((dstp >= r * _HR) & (dstp < (r + 1) * _HR),
                   dstp - r * _HR, _LDUMMY) for r in range(2)])
    tipo3 = jnp.pad(tipo_ids, (0, _NW * 384 - _N)).reshape(_NW, 3, 128)
    mask2d = jnp.pad(mask_embed, (0, _NP - _N)).reshape(80, 128)

    _emb_deg, _prop = _sc_kernels()
    dstd = dst.reshape(_NW, _E // _NW)
    xg, degp = _emb_deg(emb_table, tipo3, dstd)
    xg = xg[: _NP]
    dinv2d, rs1 = _scale(degp.reshape(_NW, 80, 128), mask2d)
    z1 = _mm1(xg, W1, rs1)
    acc1 = _prop(z1.reshape(_NC * _NP, 128), srcv_all, dstp_both)
    z2 = _mm2(acc1, z1, dinv2d, W2)
    acc2 = _prop(z2.reshape(_NC * _NP, 128), srcv_all, dstp_both)
    out = _fin(acc2, z2, dinv2d)
    return out[:_N]
